# Initial kernel scaffold; baseline (speedup 1.0000x reference)
import jax
import jax.numpy as jnp
from jax.experimental import pallas as pl


def kernel(x, edge_index, edge_type, W_sl, b_sl, W1, b1, W2, b2):
    raise NotImplementedError("write your pallas kernel here")


# speedup vs baseline: 2.3886x; 2.3886x over previous
import functools

import jax
import jax.numpy as jnp
from jax import lax
from jax.experimental import pallas as pl
from jax.experimental.pallas import tpu as pltpu
from jax.experimental.pallas import tpu_sc as plsc

NUM_REL = 4
LANES = 16
NCORES = 2
NTILES = 16
CHUNK = 128
GRP = 8
ACC_ROWS = 10240
TRASH = 10100
ZROWS = 16


def _sc_agg(x, src, dst, etype):
    N, D = x.shape
    E = src.shape[0]
    assert N <= TRASH < ACC_ROWS and ACC_ROWS % (NTILES * ZROWS) == 0
    ch = -(-E // (NTILES * CHUNK * GRP)) * GRP
    epad = NTILES * ch * CHUNK
    if epad > E:
        pad = epad - E
        src = jnp.concatenate([src, jnp.zeros((pad,), jnp.int32)])
        dst = jnp.concatenate([dst, jnp.zeros((pad,), jnp.int32)])
        etype = jnp.concatenate([etype, jnp.full((pad,), -1, jnp.int32)])
    src_m = src.reshape(NTILES, ch, CHUNK)
    dst_m = dst.reshape(NTILES, ch, CHUNK)
    type_m = etype.reshape(NTILES, ch, CHUNK)

    rows_main = N // (8 * NTILES) * 8
    rows_rem = N - rows_main * NTILES

    mesh = plsc.VectorSubcoreMesh(core_axis_name="c", subcore_axis_name="s")

    @functools.partial(
        pl.kernel,
        out_type=jax.ShapeDtypeStruct((NUM_REL * N, D), jnp.float32),
        mesh=mesh,
        scratch_types=[
            pltpu.VMEM_SHARED((ACC_ROWS, D), jnp.float32),
            pltpu.VMEM((GRP, CHUNK), jnp.int32),
            pltpu.VMEM((GRP, CHUNK), jnp.int32),
            pltpu.VMEM((GRP, CHUNK), jnp.int32),
            pltpu.VMEM((GRP, CHUNK), jnp.int32),
            pltpu.VMEM((2, CHUNK, D), jnp.float32),
            pltpu.VMEM((ZROWS, D), jnp.float32),
            pltpu.SemaphoreType.DMA,
            pltpu.SemaphoreType.DMA,
        ],
    )
    def body(x_hbm, src_hbm, dst_hbm, type_hbm, out_hbm,
             acc, src_g, dst_g, type_g, sidx_g, rows, zbuf, sem0, sem1):
        cid = lax.axis_index("c")
        tid = lax.axis_index("s")
        sems = (sem0, sem1)

        @pl.loop(0, ZROWS)
        def _(i):
            z = jnp.zeros((LANES,), jnp.float32)
            for j in range(D // LANES):
                zbuf[i, pl.ds(j * LANES, LANES)] = z

        def start(k, buf):
            pltpu.async_copy(x_hbm.at[src_g.at[k]], rows.at[buf], sems[buf])

        def wait(buf):
            pltpu.make_async_copy(x_hbm.at[src_g.at[0]], rows.at[buf],
                                  sems[buf]).wait()

        def scatter(k, buf):
            pltpu.sync_copy(rows.at[buf], acc.at[sidx_g.at[k]], add=True)

        zrows_tile = ACC_ROWS // NTILES

        for p in range(NUM_REL // NCORES):
            rel = (NUM_REL // NCORES) * cid + p

            @pl.loop(0, zrows_tile // ZROWS)
            def _(i):
                pltpu.sync_copy(
                    zbuf, acc.at[pl.ds(tid * zrows_tile + i * ZROWS, ZROWS)])

            plsc.subcore_barrier()

            @pl.loop(0, ch // GRP)
            def _(grp):
                gb = grp * GRP
                pltpu.sync_copy(src_hbm.at[tid, pl.ds(gb, GRP)], src_g)
                pltpu.sync_copy(dst_hbm.at[tid, pl.ds(gb, GRP)], dst_g)
                pltpu.sync_copy(type_hbm.at[tid, pl.ds(gb, GRP)], type_g)
                for k in range(GRP):
                    for j in range(CHUNK // LANES):
                        sl = pl.ds(j * LANES, LANES)
                        sidx_g[k, sl] = jnp.where(type_g[k, sl] == rel,
                                                  dst_g[k, sl], TRASH)
                start(0, 0)
                for k in range(GRP):
                    if k + 1 < GRP:
                        start(k + 1, (k + 1) % 2)
                    wait(k % 2)
                    scatter(k, k % 2)

            plsc.subcore_barrier()

            off = pl.multiple_of(rel * N + tid * rows_main, 8)
            pltpu.sync_copy(acc.at[pl.ds(tid * rows_main, rows_main)],
                            out_hbm.at[pl.ds(off, rows_main)])
            if rows_rem:
                @pl.when(tid == 0)
                def _():
                    off2 = pl.multiple_of(rel * N + rows_main * NTILES, 8)
                    pltpu.sync_copy(
                        acc.at[pl.ds(rows_main * NTILES, rows_rem)],
                        out_hbm.at[pl.ds(off2, rows_rem)])

            plsc.subcore_barrier()

    return body(x, src_m, dst_m, type_m)


def _tc_body(x_ref, agg_ref, wsl_ref, bsl_ref, w1_ref, b1_ref, w2_ref, b2_ref,
             o_ref):
    x = x_ref[...]
    acc = jnp.dot(x, wsl_ref[...], preferred_element_type=jnp.float32)
    acc = acc + bsl_ref[...]
    for i in range(NUM_REL):
        h = x + agg_ref[i]
        t = jnp.dot(h, w1_ref[i], preferred_element_type=jnp.float32)
        t = jnp.maximum(t + b1_ref[i], 0.0)
        acc = acc + jnp.dot(t, w2_ref[i], preferred_element_type=jnp.float32)
        acc = acc + b2_ref[i]
    o_ref[...] = acc


def _tc_mlp(x, agg, W_sl, b_sl, W1, b1, W2, b2):
    N, D = x.shape
    bn = 400
    assert N % bn == 0
    return pl.pallas_call(
        _tc_body,
        grid=(N // bn,),
        in_specs=[
            pl.BlockSpec((bn, D), lambda i: (i, 0)),
            pl.BlockSpec((NUM_REL, bn, D), lambda i: (0, i, 0)),
            pl.BlockSpec((D, D), lambda i: (0, 0)),
            pl.BlockSpec((1, D), lambda i: (0, 0)),
            pl.BlockSpec((NUM_REL, D, D), lambda i: (0, 0, 0)),
            pl.BlockSpec((NUM_REL, 1, D), lambda i: (0, 0, 0)),
            pl.BlockSpec((NUM_REL, D, D), lambda i: (0, 0, 0)),
            pl.BlockSpec((NUM_REL, 1, D), lambda i: (0, 0, 0)),
        ],
        out_specs=pl.BlockSpec((bn, D), lambda i: (i, 0)),
        out_shape=jax.ShapeDtypeStruct((N, D), jnp.float32),
    )(x, agg, W_sl, b_sl.reshape(1, D), W1, b1.reshape(NUM_REL, 1, D),
      W2, b2.reshape(NUM_REL, 1, D))


def kernel(x, edge_index, edge_type, W_sl, b_sl, W1, b1, W2, b2):
    N, D = x.shape
    src = edge_index[0].astype(jnp.int32)
    dst = edge_index[1].astype(jnp.int32)
    et = edge_type.astype(jnp.int32)
    agg = _sc_agg(x, src, dst, et).reshape(NUM_REL, N, D)
    return _tc_mlp(x, agg, W_sl, b_sl, W1, b1, W2, b2)
